# baseline (speedup 1.0000x reference)
import jax
import jax.numpy as jnp
from jax.experimental import pallas as pl

DIM = 768
NHEAD = 12
HEAD_DIM = DIM // NHEAD
TOPK = 40
REGION = 2
EPS = 1e-5
SCALE = DIM ** (-0.5)
NEG = -1e30


def _bdot(a, b, dn):
    return jax.lax.dot_general(a, b, dn, preferred_element_type=jnp.float32)


_DN_NN = (((1,), (0,)), ((), ()))
_DN_NT = (((1,), (1,)), ((), ()))


def _cat3(x):
    z = jnp.zeros((1, x.shape[1]), x.dtype)
    xm1 = jnp.concatenate([z, x[:-1]], axis=0)
    xp1 = jnp.concatenate([x[1:], z], axis=0)
    return jnp.concatenate([xm1, x, xp1], axis=1)


def _qkv_kernel(xT_ref, wq_ref, wk_ref, wv_ref,
                mq_ref, sq_ref, bq_ref, mk_ref, sk_ref, bk_ref,
                qs_ref, ks_ref, v_ref, qrg_ref, krg_ref):
    xh = xT_ref[0].astype(jnp.bfloat16)
    xch = _cat3(xh)
    qbn = (_bdot(xch, wq_ref[...], _DN_NN) - mq_ref[...]) * sq_ref[...] + bq_ref[...]
    kbn = (_bdot(xch, wk_ref[...], _DN_NN) - mk_ref[...]) * sk_ref[...] + bk_ref[...]
    v_ref[0] = _bdot(xh, wv_ref[...], _DN_NN)
    qs_ref[0] = (qbn * SCALE).astype(jnp.bfloat16)
    ks_ref[0] = kbn.astype(jnp.bfloat16)
    T, C = qbn.shape
    R = T // REGION
    qrg_ref[0] = jnp.mean(qbn.reshape(R, REGION, C), axis=1)
    krg_ref[0] = jnp.mean(kbn.reshape(R, REGION, C), axis=1)


def _routing_kernel(qrg_ref, krg_ref, e2_ref, bias_ref):
    N, R, C = qrg_ref.shape
    a_list = [
        _bdot(qrg_ref[n].astype(jnp.bfloat16),
              krg_ref[n].astype(jnp.bfloat16), _DN_NT)
        for n in range(N)
    ]
    a_all = jnp.concatenate(a_list, axis=0)
    lane = jax.lax.broadcasted_iota(jnp.int32, a_all.shape, 1)

    def body(_, a):
        m = jnp.max(a, axis=1, keepdims=True)
        cand = jnp.where(a == m, lane, R)
        mi = jnp.min(cand, axis=1, keepdims=True)
        return jnp.where(lane == mi, NEG, a)

    a_left = jax.lax.fori_loop(0, TOPK, body, a_all)
    sel = (a_left == NEG).astype(jnp.bfloat16)

    e2 = e2_ref[...]
    m_rt = _bdot(sel, e2, _DN_NN)
    for n in range(N):
        mt = _bdot(e2, m_rt[n * R:(n + 1) * R].astype(jnp.bfloat16),
                   (((0,), (0,)), ((), ())))
        bias_ref[n] = ((mt - 1.0) * (-NEG)).astype(jnp.bfloat16)


HEADS_PER_STEP = 2


def _attn_kernel(q_ref, k_ref, v_ref, b_ref, o_ref):
    b = b_ref[0].astype(jnp.float32)
    for i in range(HEADS_PER_STEP):
        s = _bdot(q_ref[0, i], k_ref[0, i], _DN_NT)
        s = s + b
        m = jnp.max(s, axis=1, keepdims=True)
        e = jnp.exp(s - m)
        p = e / jnp.sum(e, axis=1, keepdims=True)
        o_ref[0, i] = _bdot(p.astype(jnp.bfloat16),
                            v_ref[0, i].astype(jnp.bfloat16), _DN_NN)


def _out_kernel(a_ref, v_ref, wl3_ref, bl_ref, woh_ref, bo_ref, o_ref):
    v = v_ref[0]
    C = v.shape[1]
    zrow = jnp.zeros((1, C), v.dtype)
    vm1 = jnp.concatenate([zrow, v[:-1]], axis=0)
    vp1 = jnp.concatenate([v[1:], zrow], axis=0)
    wl3 = wl3_ref[...]
    lepe = vm1 * wl3[0:1] + v * wl3[1:2] + vp1 * wl3[2:3] + bl_ref[...]
    y = a_ref[0] + lepe
    o_ref[0] = _bdot(y.astype(jnp.bfloat16), woh_ref[...], _DN_NN) + bo_ref[...]


@jax.jit
def kernel(x, w_q, g_q, b_q, m_q, v_q, w_k, g_k, b_k, m_k, v_k,
           w_v, w_lepe, b_lepe, w_out, b_out):
    N, C, T = x.shape
    R = T // REGION
    f32 = jnp.float32

    xT = jnp.transpose(x, (0, 2, 1))

    s_q = g_q * jax.lax.rsqrt(v_q + EPS)
    s_k = g_k * jax.lax.rsqrt(v_k + EPS)
    wq_h = w_q.transpose(2, 1, 0).reshape(3 * C, C).astype(jnp.bfloat16)
    wk_h = w_k.transpose(2, 1, 0).reshape(3 * C, C).astype(jnp.bfloat16)
    wv_h = w_v[:, :, 0].T.astype(jnp.bfloat16)
    wo_h = w_out[:, :, 0].T.astype(jnp.bfloat16)
    wl3 = w_lepe[:, 0, :].T
    bl2 = b_lepe[None, :]
    bo2 = b_out[None, :]
    e2 = jnp.repeat(jnp.eye(R, dtype=jnp.bfloat16), REGION, axis=1)

    vec = pl.BlockSpec((1, C), lambda n: (0, 0))
    qs, ks, v, qrg, krg = pl.pallas_call(
        _qkv_kernel,
        grid=(N,),
        in_specs=[
            pl.BlockSpec((1, T, C), lambda n: (n, 0, 0)),
            pl.BlockSpec((3 * C, C), lambda n: (0, 0)),
            pl.BlockSpec((3 * C, C), lambda n: (0, 0)),
            pl.BlockSpec((C, C), lambda n: (0, 0)),
            vec, vec, vec, vec, vec, vec,
        ],
        out_specs=[
            pl.BlockSpec((1, T, C), lambda n: (n, 0, 0)),
            pl.BlockSpec((1, T, C), lambda n: (n, 0, 0)),
            pl.BlockSpec((1, T, C), lambda n: (n, 0, 0)),
            pl.BlockSpec((1, R, C), lambda n: (n, 0, 0)),
            pl.BlockSpec((1, R, C), lambda n: (n, 0, 0)),
        ],
        out_shape=[
            jax.ShapeDtypeStruct((N, T, C), jnp.bfloat16),
            jax.ShapeDtypeStruct((N, T, C), jnp.bfloat16),
            jax.ShapeDtypeStruct((N, T, C), f32),
            jax.ShapeDtypeStruct((N, R, C), f32),
            jax.ShapeDtypeStruct((N, R, C), f32),
        ],
    )(xT, wq_h, wk_h, wv_h,
      m_q[None, :], s_q[None, :], b_q[None, :],
      m_k[None, :], s_k[None, :], b_k[None, :])

    bias = pl.pallas_call(
        _routing_kernel,
        in_specs=[
            pl.BlockSpec((N, R, C), lambda: (0, 0, 0)),
            pl.BlockSpec((N, R, C), lambda: (0, 0, 0)),
            pl.BlockSpec((R, T), lambda: (0, 0)),
        ],
        out_specs=pl.BlockSpec((N, T, T), lambda: (0, 0, 0)),
        out_shape=jax.ShapeDtypeStruct((N, T, T), jnp.bfloat16),
    )(qrg, krg, e2)

    def to_heads(t):
        return t.reshape(N, T, NHEAD, HEAD_DIM).transpose(0, 2, 1, 3)

    qh, kh, vh = to_heads(qs), to_heads(ks), to_heads(v)

    attn = pl.pallas_call(
        _attn_kernel,
        grid=(N, NHEAD // HEADS_PER_STEP),
        in_specs=[
            pl.BlockSpec((1, HEADS_PER_STEP, T, HEAD_DIM), lambda n, h: (n, h, 0, 0)),
            pl.BlockSpec((1, HEADS_PER_STEP, T, HEAD_DIM), lambda n, h: (n, h, 0, 0)),
            pl.BlockSpec((1, HEADS_PER_STEP, T, HEAD_DIM), lambda n, h: (n, h, 0, 0)),
            pl.BlockSpec((1, T, T), lambda n, h: (n, 0, 0)),
        ],
        out_specs=pl.BlockSpec((1, HEADS_PER_STEP, T, HEAD_DIM), lambda n, h: (n, h, 0, 0)),
        out_shape=jax.ShapeDtypeStruct((N, NHEAD, T, HEAD_DIM), f32),
    )(qh, kh, vh, bias)

    attn = attn.transpose(0, 2, 1, 3).reshape(N, T, C)

    outT = pl.pallas_call(
        _out_kernel,
        grid=(N,),
        in_specs=[
            pl.BlockSpec((1, T, C), lambda n: (n, 0, 0)),
            pl.BlockSpec((1, T, C), lambda n: (n, 0, 0)),
            pl.BlockSpec((3, C), lambda n: (0, 0)),
            pl.BlockSpec((1, C), lambda n: (0, 0)),
            pl.BlockSpec((C, C), lambda n: (0, 0)),
            pl.BlockSpec((1, C), lambda n: (0, 0)),
        ],
        out_specs=pl.BlockSpec((1, T, C), lambda n: (n, 0, 0)),
        out_shape=jax.ShapeDtypeStruct((N, T, C), f32),
    )(attn, v, wl3, bl2, wo_h, bo2)

    return jnp.transpose(outT, (0, 2, 1))


# speedup vs baseline: 1.0555x; 1.0555x over previous
import jax
import jax.numpy as jnp
from jax.experimental import pallas as pl

DIM = 768
NHEAD = 12
HEAD_DIM = DIM // NHEAD
TOPK = 40
REGION = 2
EPS = 1e-5
SCALE = DIM ** (-0.5)
NEG = -1e30


def _bdot(a, b, dn):
    return jax.lax.dot_general(a, b, dn, preferred_element_type=jnp.float32)


_DN_NN = (((1,), (0,)), ((), ()))
_DN_NT = (((1,), (1,)), ((), ()))


def _cat3(x):
    z = jnp.zeros((1, x.shape[1]), x.dtype)
    xm1 = jnp.concatenate([z, x[:-1]], axis=0)
    xp1 = jnp.concatenate([x[1:], z], axis=0)
    return jnp.concatenate([xm1, x, xp1], axis=1)


def _qkv_routing_kernel(xT_ref, wq_ref, wk_ref, wv_ref,
                        mq_ref, sq_ref, bq_ref, mk_ref, sk_ref, bk_ref,
                        e2_ref, qs_ref, ks_ref, v_ref, bias_ref):
    xh = xT_ref[0].astype(jnp.bfloat16)
    xch = _cat3(xh)
    qbn = (_bdot(xch, wq_ref[...], _DN_NN) - mq_ref[...]) * sq_ref[...] + bq_ref[...]
    kbn = (_bdot(xch, wk_ref[...], _DN_NN) - mk_ref[...]) * sk_ref[...] + bk_ref[...]
    v_ref[0] = _bdot(xh, wv_ref[...], _DN_NN)
    qs_ref[0] = (qbn * SCALE).astype(jnp.bfloat16)
    ks_ref[0] = kbn.astype(jnp.bfloat16)

    T, C = qbn.shape
    R = T // REGION
    q_rg = jnp.mean(qbn.reshape(R, REGION, C), axis=1)
    k_rg = jnp.mean(kbn.reshape(R, REGION, C), axis=1)
    a_r = _bdot(q_rg.astype(jnp.bfloat16),
                k_rg.astype(jnp.bfloat16), _DN_NT)
    e2 = e2_ref[...]

    lane = jax.lax.broadcasted_iota(jnp.int32, (R, R), 1)

    def body(_, a):
        m = jnp.max(a, axis=1, keepdims=True)
        cand = jnp.where(a == m, lane, R)
        mi = jnp.min(cand, axis=1, keepdims=True)
        return jnp.where(lane == mi, NEG, a)

    a_left = jax.lax.fori_loop(0, TOPK, body, a_r)
    sel = (a_left == NEG).astype(jnp.float32)

    m_rt = _bdot(sel.astype(jnp.bfloat16), e2, _DN_NN)
    mt = _bdot(e2, m_rt.astype(jnp.bfloat16),
               (((0,), (0,)), ((), ())))
    bias_ref[0] = ((mt - 1.0) * (-NEG)).astype(jnp.bfloat16)


HEADS_PER_STEP = 2


def _attn_kernel(q_ref, k_ref, v_ref, b_ref, o_ref):
    b = b_ref[0].astype(jnp.float32)
    for i in range(HEADS_PER_STEP):
        s = _bdot(q_ref[0, i], k_ref[0, i], _DN_NT)
        s = s + b
        m = jnp.max(s, axis=1, keepdims=True)
        e = jnp.exp(s - m)
        p = e / jnp.sum(e, axis=1, keepdims=True)
        o_ref[0, i] = _bdot(p.astype(jnp.bfloat16),
                            v_ref[0, i].astype(jnp.bfloat16), _DN_NN)


def _out_kernel(a_ref, v_ref, wl3_ref, bl_ref, woh_ref, bo_ref, o_ref):
    v = v_ref[0]
    C = v.shape[1]
    zrow = jnp.zeros((1, C), v.dtype)
    vm1 = jnp.concatenate([zrow, v[:-1]], axis=0)
    vp1 = jnp.concatenate([v[1:], zrow], axis=0)
    wl3 = wl3_ref[...]
    lepe = vm1 * wl3[0:1] + v * wl3[1:2] + vp1 * wl3[2:3] + bl_ref[...]
    y = a_ref[0] + lepe
    o_ref[0] = _bdot(y.astype(jnp.bfloat16), woh_ref[...], _DN_NN) + bo_ref[...]


@jax.jit
def kernel(x, w_q, g_q, b_q, m_q, v_q, w_k, g_k, b_k, m_k, v_k,
           w_v, w_lepe, b_lepe, w_out, b_out):
    N, C, T = x.shape
    R = T // REGION
    f32 = jnp.float32

    xT = jnp.transpose(x, (0, 2, 1))

    s_q = g_q * jax.lax.rsqrt(v_q + EPS)
    s_k = g_k * jax.lax.rsqrt(v_k + EPS)
    wq_h = w_q.transpose(2, 1, 0).reshape(3 * C, C).astype(jnp.bfloat16)
    wk_h = w_k.transpose(2, 1, 0).reshape(3 * C, C).astype(jnp.bfloat16)
    wv_h = w_v[:, :, 0].T.astype(jnp.bfloat16)
    wo_h = w_out[:, :, 0].T.astype(jnp.bfloat16)
    wl3 = w_lepe[:, 0, :].T
    bl2 = b_lepe[None, :]
    bo2 = b_out[None, :]
    e2 = jnp.repeat(jnp.eye(R, dtype=jnp.bfloat16), REGION, axis=1)

    vec = pl.BlockSpec((1, C), lambda n: (0, 0))
    qs, ks, v, bias = pl.pallas_call(
        _qkv_routing_kernel,
        grid=(N,),
        in_specs=[
            pl.BlockSpec((1, T, C), lambda n: (n, 0, 0)),
            pl.BlockSpec((3 * C, C), lambda n: (0, 0)),
            pl.BlockSpec((3 * C, C), lambda n: (0, 0)),
            pl.BlockSpec((C, C), lambda n: (0, 0)),
            vec, vec, vec, vec, vec, vec,
            pl.BlockSpec((R, T), lambda n: (0, 0)),
        ],
        out_specs=[
            pl.BlockSpec((1, T, C), lambda n: (n, 0, 0)),
            pl.BlockSpec((1, T, C), lambda n: (n, 0, 0)),
            pl.BlockSpec((1, T, C), lambda n: (n, 0, 0)),
            pl.BlockSpec((1, T, T), lambda n: (n, 0, 0)),
        ],
        out_shape=[
            jax.ShapeDtypeStruct((N, T, C), jnp.bfloat16),
            jax.ShapeDtypeStruct((N, T, C), jnp.bfloat16),
            jax.ShapeDtypeStruct((N, T, C), f32),
            jax.ShapeDtypeStruct((N, T, T), jnp.bfloat16),
        ],
    )(xT, wq_h, wk_h, wv_h,
      m_q[None, :], s_q[None, :], b_q[None, :],
      m_k[None, :], s_k[None, :], b_k[None, :], e2)

    def to_heads(t):
        return t.reshape(N, T, NHEAD, HEAD_DIM).transpose(0, 2, 1, 3)

    qh, kh, vh = to_heads(qs), to_heads(ks), to_heads(v)

    attn = pl.pallas_call(
        _attn_kernel,
        grid=(N, NHEAD // HEADS_PER_STEP),
        in_specs=[
            pl.BlockSpec((1, HEADS_PER_STEP, T, HEAD_DIM), lambda n, h: (n, h, 0, 0)),
            pl.BlockSpec((1, HEADS_PER_STEP, T, HEAD_DIM), lambda n, h: (n, h, 0, 0)),
            pl.BlockSpec((1, HEADS_PER_STEP, T, HEAD_DIM), lambda n, h: (n, h, 0, 0)),
            pl.BlockSpec((1, T, T), lambda n, h: (n, 0, 0)),
        ],
        out_specs=pl.BlockSpec((1, HEADS_PER_STEP, T, HEAD_DIM), lambda n, h: (n, h, 0, 0)),
        out_shape=jax.ShapeDtypeStruct((N, NHEAD, T, HEAD_DIM), f32),
    )(qh, kh, vh, bias)

    attn = attn.transpose(0, 2, 1, 3).reshape(N, T, C)

    outT = pl.pallas_call(
        _out_kernel,
        grid=(N,),
        in_specs=[
            pl.BlockSpec((1, T, C), lambda n: (n, 0, 0)),
            pl.BlockSpec((1, T, C), lambda n: (n, 0, 0)),
            pl.BlockSpec((3, C), lambda n: (0, 0)),
            pl.BlockSpec((1, C), lambda n: (0, 0)),
            pl.BlockSpec((C, C), lambda n: (0, 0)),
            pl.BlockSpec((1, C), lambda n: (0, 0)),
        ],
        out_specs=pl.BlockSpec((1, T, C), lambda n: (n, 0, 0)),
        out_shape=jax.ShapeDtypeStruct((N, T, C), f32),
    )(attn, v, wl3, bl2, wo_h, bo2)

    return jnp.transpose(outT, (0, 2, 1))
